# baseline (speedup 1.0000x reference)
import functools

import jax
import jax.numpy as jnp
from jax import lax
from jax.experimental import pallas as pl
from jax.experimental.pallas import tpu as pltpu
from jax.experimental.pallas import tpu_sc as plsc

DIM = 128
RANK = 64

NC = 2
NS = 16
NW = NC * NS


def _mm_body(ut_ref, v_ref, e_ref):
    e_ref[...] = lax.dot_general(
        ut_ref[...].astype(jnp.bfloat16), v_ref[...].astype(jnp.bfloat16),
        dimension_numbers=(((0,), (0,)), ((), ())),
        preferred_element_type=jnp.float32,
        precision=lax.Precision.DEFAULT,
    )


@functools.partial(jax.jit, static_argnames=("blk",))
def _compute_table(U, V, blk=25088):
    vocab = U.shape[0]
    vocab_pad = ((vocab + blk - 1) // blk) * blk
    return pl.pallas_call(
        _mm_body,
        grid=(vocab_pad // blk,),
        in_specs=[
            pl.BlockSpec((RANK, blk), lambda i: (0, i)),
            pl.BlockSpec((RANK, DIM), lambda i: (0, 0)),
        ],
        out_specs=pl.BlockSpec((blk, DIM), lambda i: (i, 0)),
        out_shape=jax.ShapeDtypeStruct((vocab_pad, DIM), jnp.float32),
    )(U.T, V)


@functools.lru_cache(maxsize=None)
def _make_gather(n_tokens, chunk, nbuf):
    n_rows_idx = n_tokens // chunk
    n_chunks = n_rows_idx // NW
    assert n_tokens % (NW * chunk) == 0 and n_chunks % nbuf == 0
    n_outer = n_chunks // nbuf

    mesh = plsc.VectorSubcoreMesh(core_axis_name="c", subcore_axis_name="s")
    scratch = [
        pltpu.VMEM((n_chunks, chunk), jnp.int32),
        pltpu.VMEM((nbuf, chunk, DIM), jnp.float32),
    ]
    scratch += [pltpu.SemaphoreType.DMA] * (2 * nbuf)

    @functools.partial(
        pl.kernel,
        out_type=jax.ShapeDtypeStruct((n_tokens, DIM), jnp.float32),
        mesh=mesh,
        scratch_types=scratch,
    )
    def gather_kernel(table_hbm, idx_hbm, out_hbm, idx_v, rows_v, *sems):
        gs = sems[:nbuf]
        ws = sems[nbuf:]
        wid = lax.axis_index("s") * NC + lax.axis_index("c")
        rbase = wid * n_chunks
        tbase = rbase * chunk

        def start_gather(g, slot):
            pltpu.async_copy(table_hbm.at[idx_v.at[g]], rows_v.at[slot], gs[slot])

        def wait_gather(slot):
            pltpu.make_async_copy(
                table_hbm.at[idx_v.at[0]], rows_v.at[slot], gs[slot]
            ).wait()

        def start_write(g, slot):
            pltpu.async_copy(
                rows_v.at[slot], out_hbm.at[pl.ds(tbase + g * chunk, chunk)],
                ws[slot],
            )

        def wait_write(slot):
            pltpu.make_async_copy(
                rows_v.at[slot], out_hbm.at[pl.ds(tbase, chunk)], ws[slot]
            ).wait()

        pltpu.sync_copy(idx_hbm.at[wid], idx_v)
        for b in range(nbuf - 1):
            start_gather(b, b)

        def outer(j, carry):
            for b in range(nbuf):
                s = b
                sp = (b - 1) % nbuf
                g = j * nbuf + b
                gp = g + nbuf - 1
                if b == 0:
                    @pl.when(j > 0)
                    def _():
                        wait_write(sp)
                    start_gather(gp, sp)
                else:
                    @pl.when(j < n_outer - 1)
                    def _():
                        wait_write(sp)
                        start_gather(gp, sp)
                wait_gather(s)
                start_write(g, s)
            return carry

        lax.fori_loop(0, n_outer, outer, 0)
        for s in range(nbuf):
            wait_write(s)

    return gather_kernel


def kernel(token_ids, U, V):
    B, L = token_ids.shape
    n_tokens = B * L
    chunk = 80
    ids = token_ids.reshape(NW, n_tokens // (NW * chunk), chunk).astype(jnp.int32)
    table = _compute_table(U, V)
    out = _make_gather(n_tokens, chunk, 10)(table, ids)
    return out.reshape(B, L, DIM)
